# baseline (speedup 1.0000x reference)
import functools

import jax
import jax.numpy as jnp
from jax.experimental import pallas as pl
from jax.experimental.pallas import tpu as pltpu

_TN = 512


def _tbcnn_block(nodes_ref, mb_ref, wl_ref, wr_ref, wt_ref, b_ref, out_ref, *, c, h):
    s = mb_ref[:, 0:h]
    a = jnp.zeros_like(s)
    for i in range(1, c):
        x = mb_ref[:, i * h:(i + 1) * h]
        s = s + x
        a = a + (i / (c - 1)) * x
    wl = wl_ref[...]
    acc = jnp.dot(s, wl, preferred_element_type=jnp.float32)
    acc += jnp.dot(a, wr_ref[...] - wl, preferred_element_type=jnp.float32)
    acc += jnp.dot(nodes_ref[...], wt_ref[...], preferred_element_type=jnp.float32)
    out_ref[...] = jnp.maximum(acc + b_ref[...], 0.0)


def kernel(nodes_h, mailbox_h, W_left, W_right, W_top, b_conv):
    n, c, h = mailbox_h.shape
    return pl.pallas_call(
        functools.partial(_tbcnn_block, c=c, h=h),
        grid=(pl.cdiv(n, _TN),),
        in_specs=[
            pl.BlockSpec((_TN, h), lambda i: (i, 0)),
            pl.BlockSpec((_TN, c * h), lambda i: (i, 0)),
            pl.BlockSpec((h, h), lambda i: (0, 0)),
            pl.BlockSpec((h, h), lambda i: (0, 0)),
            pl.BlockSpec((h, h), lambda i: (0, 0)),
            pl.BlockSpec((1, h), lambda i: (0, 0)),
        ],
        out_specs=pl.BlockSpec((_TN, h), lambda i: (i, 0)),
        out_shape=jax.ShapeDtypeStruct((n, h), jnp.float32),
        compiler_params=pltpu.CompilerParams(
            dimension_semantics=("parallel",),
        ),
    )(nodes_h, mailbox_h.reshape(n, c * h), W_left, W_right, W_top, b_conv)


# speedup vs baseline: 2.7371x; 2.7371x over previous
import functools

import jax
import jax.numpy as jnp
from jax.experimental import pallas as pl
from jax.experimental.pallas import tpu as pltpu

_TN = 512


def _tbcnn_block(nodes_ref, mb0_ref, mb1_ref, wl_ref, wr_ref, wt_ref, b_ref,
                 out_ref, *, c, h):
    half = c // 2
    x0 = mb0_ref[...]
    x1 = mb1_ref[...]
    inv = 1.0 / (c - 1)
    cf = jax.lax.broadcasted_iota(jnp.int32, (1, half, 1), 1).astype(jnp.float32)
    s = jnp.sum(x0 + x1, axis=1)
    a = jnp.sum((cf * inv) * (x0 + x1) + (half * inv) * x1, axis=1)
    wl = wl_ref[...]
    acc = jnp.dot(s, wl, preferred_element_type=jnp.float32)
    acc += jnp.dot(a, wr_ref[...] - wl, preferred_element_type=jnp.float32)
    acc += jnp.dot(nodes_ref[...], wt_ref[...], preferred_element_type=jnp.float32)
    out_ref[...] = jnp.maximum(acc + b_ref[...], 0.0)


def kernel(nodes_h, mailbox_h, W_left, W_right, W_top, b_conv):
    n, c, h = mailbox_h.shape
    return pl.pallas_call(
        functools.partial(_tbcnn_block, c=c, h=h),
        grid=(pl.cdiv(n, _TN),),
        in_specs=[
            pl.BlockSpec((_TN, h), lambda i: (i, 0)),
            pl.BlockSpec((_TN, c // 2, h), lambda i: (i, 0, 0)),
            pl.BlockSpec((_TN, c // 2, h), lambda i: (i, 1, 0)),
            pl.BlockSpec((h, h), lambda i: (0, 0)),
            pl.BlockSpec((h, h), lambda i: (0, 0)),
            pl.BlockSpec((h, h), lambda i: (0, 0)),
            pl.BlockSpec((1, h), lambda i: (0, 0)),
        ],
        out_specs=pl.BlockSpec((_TN, h), lambda i: (i, 0)),
        out_shape=jax.ShapeDtypeStruct((n, h), jnp.float32),
        compiler_params=pltpu.CompilerParams(
            dimension_semantics=("parallel",),
        ),
    )(nodes_h, mailbox_h, mailbox_h, W_left, W_right, W_top, b_conv)
